# Initial kernel scaffold; baseline (speedup 1.0000x reference)
import jax
import jax.numpy as jnp
from jax.experimental import pallas as pl


def kernel(logits, k, p):
    raise NotImplementedError("write your pallas kernel here")


# speedup vs baseline: 24.7915x; 24.7915x over previous
import jax
import jax.numpy as jnp
from jax import lax
from jax.experimental import pallas as pl
from jax.experimental.pallas import tpu as pltpu

_V = 100000
_SUB = 784
_VP = _SUB * 128


def _row_kernel(k_ref, p_ref, x_ref, q_ref, o_ref):
    i = pl.program_id(0)
    x = x_ref[0]
    q = q_ref[0]
    kk = jnp.maximum(k_ref[i], 1)
    p = p_ref[i]

    m = jnp.max(x)
    u = lax.bitcast_convert_type(x, jnp.uint32)
    sign = u >> jnp.uint32(31)
    us = u ^ (sign * jnp.uint32(0xFFFFFFFF) | jnp.uint32(0x80000000))

    def body_c(b, prefix):
        t = prefix | jnp.left_shift(jnp.uint32(1), jnp.uint32(31) - b.astype(jnp.uint32))
        cnt = jnp.sum((us >= t).astype(jnp.int32))
        return jnp.where(cnt >= kk, t, prefix)

    c_us = lax.fori_loop(0, 32, body_c, jnp.uint32(0), unroll=True)

    e = jnp.where(us >= c_us, jnp.exp(x - m), 0.0)
    z = jnp.sum(e)
    target = p * z

    def body_t(b, prefix):
        t = prefix | jnp.left_shift(jnp.uint32(1), jnp.uint32(31) - b.astype(jnp.uint32))
        mass = jnp.sum(jnp.where(us >= t, e, 0.0))
        return jnp.where(mass >= target, t, prefix)

    theta = lax.fori_loop(0, 32, body_t, jnp.uint32(0), unroll=True)

    keep = (us >= theta) | (x == m)
    score = jnp.where(keep, e / q, -1.0)
    lin = (lax.broadcasted_iota(jnp.int32, (_SUB, 128), 0) * 128
           + lax.broadcasted_iota(jnp.int32, (_SUB, 128), 1))
    big = jnp.int32(2 ** 30)
    nan_mask = (q == 0.0) & jnp.logical_not(keep)
    first_nan = jnp.min(jnp.where(nan_mask, lin, big))
    best = jnp.max(score)
    tok = jnp.min(jnp.where(score == best, lin, big))
    o_ref[i] = jnp.where(first_nan < big, first_nan, tok)


def kernel(logits, k, p):
    b, v = logits.shape
    q = jax.random.exponential(jax.random.key(42), (b, v), dtype=jnp.float32)
    xp = jnp.pad(logits, ((0, 0), (0, _VP - v)), constant_values=-1e30)
    qp = jnp.pad(q, ((0, 0), (0, _VP - v)), constant_values=1.0)
    xp = xp.reshape(b, _SUB, 128)
    qp = qp.reshape(b, _SUB, 128)
    out = pl.pallas_call(
        _row_kernel,
        grid=(b,),
        in_specs=[
            pl.BlockSpec(memory_space=pltpu.SMEM),
            pl.BlockSpec(memory_space=pltpu.SMEM),
            pl.BlockSpec((1, _SUB, 128), lambda i: (i, 0, 0)),
            pl.BlockSpec((1, _SUB, 128), lambda i: (i, 0, 0)),
        ],
        out_specs=pl.BlockSpec(memory_space=pltpu.SMEM),
        out_shape=jax.ShapeDtypeStruct((b,), jnp.int32),
    )(k.astype(jnp.int32), p, xp, qp)
    return out


# speedup vs baseline: 41.2054x; 1.6621x over previous
import jax
import jax.numpy as jnp
from jax import lax
from jax.experimental import pallas as pl
from jax.experimental.pallas import tpu as pltpu

_V = 100000
_VP = 100352
_R = 8


def _block_kernel(k_ref, p_ref, x_ref, q_ref, o_ref):
    x = x_ref[...]
    q = q_ref[...]
    kk = jnp.maximum(k_ref[...], 1)
    pp = p_ref[...]

    m = jnp.max(x, axis=1, keepdims=True)
    u = lax.bitcast_convert_type(x, jnp.uint32)
    sign = u >> jnp.uint32(31)
    us = u ^ (sign * jnp.uint32(0xFFFFFFFF) | jnp.uint32(0x80000000))
    kkf = kk.astype(jnp.float32)

    def body_c(b, prefix):
        t = prefix | jnp.left_shift(jnp.uint32(1), jnp.uint32(31) - b.astype(jnp.uint32))
        cnt = jnp.sum((us >= t).astype(jnp.float32), axis=1, keepdims=True)
        return jnp.where(cnt >= kkf, t, prefix)

    c_us = lax.fori_loop(0, 32, body_c, jnp.zeros((_R, 1), jnp.uint32))

    e = jnp.where(us >= c_us, jnp.exp(x - m), 0.0)
    z = jnp.sum(e, axis=1, keepdims=True)
    target = pp * z

    def body_t(b, prefix):
        tb = prefix | jnp.left_shift(jnp.uint32(1), jnp.uint32(30) - b.astype(jnp.uint32))
        tf = lax.bitcast_convert_type(tb, jnp.float32)
        mass = jnp.sum(jnp.where(e >= tf, e, 0.0), axis=1, keepdims=True)
        return jnp.where(mass >= target, tb, prefix)

    theta = lax.fori_loop(0, 31, body_t, jnp.zeros((_R, 1), jnp.uint32))
    thetaf = lax.bitcast_convert_type(theta, jnp.float32)

    um = lax.bitcast_convert_type(m, jnp.uint32)
    us_max = um ^ ((um >> jnp.uint32(31)) * jnp.uint32(0xFFFFFFFF)
                   | jnp.uint32(0x80000000))
    keep = ((e >= thetaf) & (us >= c_us)) | (us == us_max)
    score = jnp.where(keep, e / q, -1.0)
    lin = lax.broadcasted_iota(jnp.int32, (_R, _VP), 1)
    big = jnp.int32(2 ** 30)
    nan_mask = (q == 0.0) & jnp.logical_not(keep)
    first_nan = jnp.min(jnp.where(nan_mask, lin, big), axis=1, keepdims=True)
    best = jnp.max(score, axis=1, keepdims=True)
    tok = jnp.min(jnp.where(score == best, lin, big), axis=1, keepdims=True)
    o_ref[...] = jnp.where(first_nan < big, first_nan, tok)


def kernel(logits, k, p):
    b, v = logits.shape
    q = jax.random.exponential(jax.random.key(42), (b, v), dtype=jnp.float32)
    xp = jnp.pad(logits, ((0, 0), (0, _VP - v)), constant_values=-1e30)
    qp = jnp.pad(q, ((0, 0), (0, _VP - v)), constant_values=1.0)
    out = pl.pallas_call(
        _block_kernel,
        grid=(b // _R,),
        in_specs=[
            pl.BlockSpec((_R, 1), lambda i: (i, 0)),
            pl.BlockSpec((_R, 1), lambda i: (i, 0)),
            pl.BlockSpec((_R, _VP), lambda i: (i, 0)),
            pl.BlockSpec((_R, _VP), lambda i: (i, 0)),
        ],
        out_specs=pl.BlockSpec((_R, 1), lambda i: (i, 0)),
        out_shape=jax.ShapeDtypeStruct((b, 1), jnp.int32),
    )(k.astype(jnp.int32).reshape(b, 1), p.reshape(b, 1), xp, qp)
    return out.reshape(-1)
